# Initial kernel scaffold; baseline (speedup 1.0000x reference)
import jax
import jax.numpy as jnp
from jax.experimental import pallas as pl


def kernel(x, atomic_numbers, batch, shift):
    raise NotImplementedError("write your pallas kernel here")


# speedup vs baseline: 21.0796x; 21.0796x over previous
import functools
import jax
import jax.numpy as jnp
from jax import lax
from jax.experimental import pallas as pl
from jax.experimental.pallas import tpu as pltpu
from jax.experimental.pallas import tpu_sc as plsc

_N_TILES = 16
_PER_TILE = 6272
_N_ATOMS_PAD = _N_TILES * _PER_TILE
_N_ITERS = _PER_TILE // 16
_N_SEG = 512

_mesh = plsc.VectorSubcoreMesh(core_axis_name="c", subcore_axis_name="s",
                               num_cores=1)


@functools.partial(
    pl.kernel,
    mesh=_mesh,
    out_type=jax.ShapeDtypeStruct((_N_SEG,), jnp.float32),
    scratch_types=[
        pltpu.VMEM((_PER_TILE,), jnp.int32),
        pltpu.VMEM((_PER_TILE,), jnp.int32),
        pltpu.VMEM((128,), jnp.float32),
        pltpu.VMEM((_N_SEG,), jnp.float32),
        pltpu.VMEM((128,), jnp.int32),
        pltpu.VMEM((128,), jnp.int32),
        pltpu.VMEM((128,), jnp.int32),
        pltpu.VMEM((128,), jnp.int32),
        pltpu.VMEM((_N_SEG,), jnp.float32),
        pltpu.VMEM((_N_SEG,), jnp.float32),
        pltpu.VMEM_SHARED((_N_SEG,), jnp.float32),
    ],
    compiler_params=pltpu.CompilerParams(needs_layout_passes=False),
)
def _shift_kernel(x_hbm, an_hbm, b_hbm, shift_hbm, out_hbm,
                  an_v, b_v, shift_v, seg_v, idx0, idx1, idx2, idx3,
                  sum_v, x_v, shared):
    wid = lax.axis_index("s")
    base = wid * _PER_TILE
    pltpu.sync_copy(an_hbm.at[pl.ds(base, _PER_TILE)], an_v)
    pltpu.sync_copy(b_hbm.at[pl.ds(base, _PER_TILE)], b_v)
    pltpu.sync_copy(shift_hbm, shift_v)

    zeros = jnp.zeros((16,), jnp.float32)
    for i in range(_N_SEG // 16):
        seg_v[pl.ds(i * 16, 16)] = zeros
    lane = lax.iota(jnp.int32, 16)
    for j, idx_ref in enumerate((idx0, idx1, idx2, idx3)):
        for v in range(8):
            idx_ref[pl.ds(v * 16, 16)] = lane + (j * 128 + v * 16)

    @pl.when(wid == 0)
    def _():
        pltpu.sync_copy(seg_v, shared)

    plsc.subcore_barrier()

    def body(i, carry):
        off = i * 16
        an16 = an_v[pl.ds(off, 16)]
        b16 = b_v[pl.ds(off, 16)]
        vals = plsc.load_gather(shift_v, [an16])
        plsc.addupdate_scatter(seg_v, [b16], vals)
        return carry

    lax.fori_loop(0, _N_ITERS, body, 0)

    for j, idx_ref in enumerate((idx0, idx1, idx2, idx3)):
        pltpu.sync_copy(seg_v.at[pl.ds(j * 128, 128)],
                        shared.at[idx_ref], add=True)

    plsc.subcore_barrier()

    @pl.when(wid == 0)
    def _():
        pltpu.sync_copy(shared, sum_v)
        pltpu.sync_copy(x_hbm, x_v)
        for i in range(_N_SEG // 16):
            sl = pl.ds(i * 16, 16)
            x_v[sl] = x_v[sl] - sum_v[sl]
        pltpu.sync_copy(x_v, out_hbm)


def kernel(x, atomic_numbers, batch, shift):
    n = atomic_numbers.shape[0]
    an_p = jnp.zeros((_N_ATOMS_PAD,), jnp.int32).at[:n].set(atomic_numbers)
    b_p = jnp.full((_N_ATOMS_PAD,), _N_SEG - 1, jnp.int32).at[:n].set(batch)
    shift_p = jnp.zeros((128,), jnp.float32).at[:shift.shape[0]].set(shift[:, 0])
    x_p = jnp.zeros((_N_SEG,), jnp.float32).at[:x.shape[0]].set(x)
    out = _shift_kernel(x_p, an_p, b_p, shift_p)
    return out[:x.shape[0]]


# speedup vs baseline: 21.7171x; 1.0302x over previous
import functools
import jax
import jax.numpy as jnp
from jax import lax
from jax.experimental import pallas as pl
from jax.experimental.pallas import tpu as pltpu
from jax.experimental.pallas import tpu_sc as plsc

_N_TILES = 16
_PER_TILE = 6272
_N_ATOMS_PAD = _N_TILES * _PER_TILE
_UNROLL = 8
_N_ITERS = _PER_TILE // (16 * _UNROLL)
_N_SEG = 512

_mesh = plsc.VectorSubcoreMesh(core_axis_name="c", subcore_axis_name="s",
                               num_cores=1)


@functools.partial(
    pl.kernel,
    mesh=_mesh,
    out_type=jax.ShapeDtypeStruct((_N_SEG,), jnp.float32),
    scratch_types=[
        pltpu.VMEM((_PER_TILE,), jnp.int32),
        pltpu.VMEM((_PER_TILE,), jnp.int32),
        pltpu.VMEM((128,), jnp.float32),
        pltpu.VMEM((_N_SEG,), jnp.float32),
        pltpu.VMEM((128,), jnp.int32),
        pltpu.VMEM((128,), jnp.int32),
        pltpu.VMEM((128,), jnp.int32),
        pltpu.VMEM((128,), jnp.int32),
        pltpu.VMEM_SHARED((_N_SEG,), jnp.float32),
        pltpu.SemaphoreType.DMA,
        pltpu.SemaphoreType.DMA,
    ],
    compiler_params=pltpu.CompilerParams(needs_layout_passes=False),
)
def _shift_kernel(x_hbm, an_hbm, b_hbm, shift_hbm, out_hbm,
                  an_v, b_v, shift_v, seg_v, idx0, idx1, idx2, idx3,
                  shared, sem_a, sem_b):
    wid = lax.axis_index("s")
    base = wid * _PER_TILE
    cp_a = pltpu.async_copy(an_hbm.at[pl.ds(base, _PER_TILE)], an_v, sem_a)
    cp_b = pltpu.async_copy(b_hbm.at[pl.ds(base, _PER_TILE)], b_v, sem_b)
    pltpu.sync_copy(shift_hbm, shift_v)

    zeros = jnp.zeros((16,), jnp.float32)
    for i in range(_N_SEG // 16):
        seg_v[pl.ds(i * 16, 16)] = zeros
    lane = lax.iota(jnp.int32, 16)
    for j, idx_ref in enumerate((idx0, idx1, idx2, idx3)):
        for v in range(8):
            idx_ref[pl.ds(v * 16, 16)] = lane + (j * 128 + v * 16)

    @pl.when(wid == 0)
    def _():
        pltpu.sync_copy(x_hbm, shared)

    cp_a.wait()
    cp_b.wait()
    plsc.subcore_barrier()

    def body(i, carry):
        off = i * (16 * _UNROLL)
        for u in range(_UNROLL):
            sl = pl.ds(off + u * 16, 16)
            an16 = an_v[sl]
            b16 = b_v[sl]
            vals = plsc.load_gather(shift_v, [an16])
            plsc.addupdate_scatter(seg_v, [b16], vals)
        return carry

    lax.fori_loop(0, _N_ITERS, body, 0)

    for i in range(_N_SEG // 16):
        sl = pl.ds(i * 16, 16)
        seg_v[sl] = zeros - seg_v[sl]

    for j, idx_ref in enumerate((idx0, idx1, idx2, idx3)):
        pltpu.sync_copy(seg_v.at[pl.ds(j * 128, 128)],
                        shared.at[idx_ref], add=True)

    plsc.subcore_barrier()

    @pl.when(wid == 0)
    def _():
        pltpu.sync_copy(shared, out_hbm)


def kernel(x, atomic_numbers, batch, shift):
    n = atomic_numbers.shape[0]
    an_p = jnp.zeros((_N_ATOMS_PAD,), jnp.int32).at[:n].set(atomic_numbers)
    b_p = jnp.full((_N_ATOMS_PAD,), _N_SEG - 1, jnp.int32).at[:n].set(batch)
    shift_p = jnp.zeros((128,), jnp.float32).at[:shift.shape[0]].set(shift[:, 0])
    x_p = jnp.zeros((_N_SEG,), jnp.float32).at[:x.shape[0]].set(x)
    out = _shift_kernel(x_p, an_p, b_p, shift_p)
    return out[:x.shape[0]]


# speedup vs baseline: 26.2796x; 1.2101x over previous
import functools
import jax
import jax.numpy as jnp
from jax import lax
from jax.experimental import pallas as pl
from jax.experimental.pallas import tpu as pltpu
from jax.experimental.pallas import tpu_sc as plsc

_N_TILES = 16
_PER_LANE = 393
_PER_TILE = 16 * _PER_LANE
_N_ATOMS_PAD = _N_TILES * _PER_TILE
_UNROLL = 8
_N_ITERS = (_PER_LANE - 1) // _UNROLL
_N_SEG = 512

_mesh = plsc.VectorSubcoreMesh(core_axis_name="c", subcore_axis_name="s",
                               num_cores=1)


@functools.partial(
    pl.kernel,
    mesh=_mesh,
    out_type=jax.ShapeDtypeStruct((_N_SEG,), jnp.float32),
    scratch_types=[
        pltpu.VMEM((_PER_TILE,), jnp.int32),
        pltpu.VMEM((_PER_TILE,), jnp.int32),
        pltpu.VMEM((128,), jnp.float32),
        pltpu.VMEM((_N_SEG,), jnp.float32),
        pltpu.VMEM((128,), jnp.int32),
        pltpu.VMEM((128,), jnp.int32),
        pltpu.VMEM((128,), jnp.int32),
        pltpu.VMEM((128,), jnp.int32),
        pltpu.VMEM_SHARED((_N_SEG,), jnp.float32),
        pltpu.SemaphoreType.DMA,
        pltpu.SemaphoreType.DMA,
    ],
    compiler_params=pltpu.CompilerParams(needs_layout_passes=False),
)
def _shift_kernel(x_hbm, an_hbm, b_hbm, shift_hbm, out_hbm,
                  an_v, b_v, shift_v, seg_v, idx0, idx1, idx2, idx3,
                  shared, sem_a, sem_b):
    wid = lax.axis_index("s")
    base = wid * _PER_TILE
    cp_a = pltpu.async_copy(an_hbm.at[pl.ds(base, _PER_TILE)], an_v, sem_a)
    cp_b = pltpu.async_copy(b_hbm.at[pl.ds(base, _PER_TILE)], b_v, sem_b)
    pltpu.sync_copy(shift_hbm, shift_v)

    zeros = jnp.zeros((16,), jnp.float32)
    for i in range(_N_SEG // 16):
        seg_v[pl.ds(i * 16, 16)] = zeros
    lane = lax.iota(jnp.int32, 16)
    for j, idx_ref in enumerate((idx0, idx1, idx2, idx3)):
        for v in range(8):
            idx_ref[pl.ds(v * 16, 16)] = lane + (j * 128 + v * 16)

    @pl.when(wid == 0)
    def _():
        pltpu.sync_copy(x_hbm, shared)

    cp_a.wait()
    cp_b.wait()
    plsc.subcore_barrier()

    lane_base = lane * _PER_LANE

    def step(i):
        idx16 = lane_base + i
        an16 = plsc.load_gather(an_v, [idx16])
        b16 = plsc.load_gather(b_v, [idx16])
        vals = plsc.load_gather(shift_v, [an16])
        plsc.addupdate_scatter(seg_v, [b16], vals)

    def body(i, carry):
        off = i * _UNROLL
        for u in range(_UNROLL):
            step(off + u)
        return carry

    lax.fori_loop(0, _N_ITERS, body, 0)
    step(_PER_LANE - 1)

    for i in range(_N_SEG // 16):
        sl = pl.ds(i * 16, 16)
        seg_v[sl] = zeros - seg_v[sl]

    for j, idx_ref in enumerate((idx0, idx1, idx2, idx3)):
        pltpu.sync_copy(seg_v.at[pl.ds(j * 128, 128)],
                        shared.at[idx_ref], add=True)

    plsc.subcore_barrier()

    @pl.when(wid == 0)
    def _():
        pltpu.sync_copy(shared, out_hbm)


def kernel(x, atomic_numbers, batch, shift):
    n = atomic_numbers.shape[0]
    an_p = jnp.zeros((_N_ATOMS_PAD,), jnp.int32).at[:n].set(atomic_numbers)
    b_p = jnp.full((_N_ATOMS_PAD,), _N_SEG - 1, jnp.int32).at[:n].set(batch)
    shift_p = jnp.zeros((128,), jnp.float32).at[:shift.shape[0]].set(shift[:, 0])
    x_p = jnp.zeros((_N_SEG,), jnp.float32).at[:x.shape[0]].set(x)
    out = _shift_kernel(x_p, an_p, b_p, shift_p)
    return out[:x.shape[0]]


# speedup vs baseline: 28.2727x; 1.0758x over previous
import functools
import jax
import jax.numpy as jnp
from jax import lax
from jax.experimental import pallas as pl
from jax.experimental.pallas import tpu as pltpu
from jax.experimental.pallas import tpu_sc as plsc

_N_ATOMS = 100000
_PL_MAIN = 393
_PL_T0 = 355
_CH_MAIN = 16 * _PL_MAIN
_CH_T0 = 16 * _PL_T0
_UNROLL = 8
_NB_MAIN = (_PL_MAIN - 1) // _UNROLL
_NB_T0 = _PL_T0 // _UNROLL
_TAIL = 3
_BUF = _CH_MAIN + 16
_N_SEG = 512

_mesh = plsc.VectorSubcoreMesh(core_axis_name="c", subcore_axis_name="s",
                               num_cores=1)


@functools.partial(
    pl.kernel,
    mesh=_mesh,
    out_type=jax.ShapeDtypeStruct((_N_SEG,), jnp.float32),
    scratch_types=[
        pltpu.VMEM((_BUF,), jnp.int32),
        pltpu.VMEM((_BUF,), jnp.int32),
        pltpu.VMEM((128,), jnp.float32),
        pltpu.VMEM((_N_SEG,), jnp.float32),
        pltpu.VMEM((128,), jnp.int32),
        pltpu.VMEM((128,), jnp.int32),
        pltpu.VMEM((128,), jnp.int32),
        pltpu.VMEM((128,), jnp.int32),
        pltpu.VMEM_SHARED((_N_SEG,), jnp.float32),
        pltpu.SemaphoreType.DMA,
        pltpu.SemaphoreType.DMA,
    ],
    compiler_params=pltpu.CompilerParams(needs_layout_passes=False),
)
def _shift_kernel(x_hbm, an_hbm, b_hbm, shift_hbm, out_hbm,
                  an_v, b_v, shift_v, seg_v, idx0, idx1, idx2, idx3,
                  shared, sem_a, sem_b):
    wid = lax.axis_index("s")
    is_t0 = wid == 0

    @pl.when(is_t0)
    def _():
        cp_a = pltpu.async_copy(an_hbm.at[pl.ds(0, _CH_T0)],
                                an_v.at[pl.ds(0, _CH_T0)], sem_a)
        cp_b = pltpu.async_copy(b_hbm.at[pl.ds(0, _CH_T0)],
                                b_v.at[pl.ds(0, _CH_T0)], sem_b)
        pltpu.sync_copy(x_hbm, shared)
        cp_a.wait()
        cp_b.wait()

    @pl.when(jnp.logical_not(is_t0))
    def _():
        base = _CH_T0 + (wid - 1) * _CH_MAIN
        cp_a = pltpu.async_copy(an_hbm.at[pl.ds(base, _CH_MAIN)],
                                an_v.at[pl.ds(0, _CH_MAIN)], sem_a)
        cp_b = pltpu.async_copy(b_hbm.at[pl.ds(base, _CH_MAIN)],
                                b_v.at[pl.ds(0, _CH_MAIN)], sem_b)
        cp_a.wait()
        cp_b.wait()

    pltpu.sync_copy(shift_hbm, shift_v)

    zeros = jnp.zeros((16,), jnp.float32)
    for i in range(_N_SEG // 16):
        seg_v[pl.ds(i * 16, 16)] = zeros
    lane = lax.iota(jnp.int32, 16)
    for j, idx_ref in enumerate((idx0, idx1, idx2, idx3)):
        for v in range(8):
            idx_ref[pl.ds(v * 16, 16)] = lane + (j * 128 + v * 16)

    plsc.subcore_barrier()

    per_lane = jnp.where(is_t0, _PL_T0, _PL_MAIN).astype(jnp.int32)
    n_blocks = jnp.where(is_t0, _NB_T0, _NB_MAIN).astype(jnp.int32)
    lane_base = lane * per_lane

    def step(i):
        idx16 = lane_base + i
        an16 = plsc.load_gather(an_v, [idx16])
        b16 = plsc.load_gather(b_v, [idx16])
        vals = plsc.load_gather(shift_v, [an16])
        plsc.addupdate_scatter(seg_v, [b16], vals)

    def body(blk, carry):
        off = blk * _UNROLL
        for u in range(_UNROLL):
            step(off + u)
        return carry

    lax.fori_loop(0, n_blocks, body, 0)

    tail_base = n_blocks * _UNROLL
    for t in range(_TAIL):
        i = tail_base + t
        mask = jnp.full((16,), i < per_lane)
        idx16 = lane_base + i
        an16 = plsc.load_gather(an_v, [idx16]) & 127
        b16 = plsc.load_gather(b_v, [idx16]) & (_N_SEG - 1)
        vals = plsc.load_gather(shift_v, [an16])
        plsc.addupdate_scatter(seg_v, [b16], vals, mask=mask)

    for i in range(_N_SEG // 16):
        sl = pl.ds(i * 16, 16)
        seg_v[sl] = zeros - seg_v[sl]

    cps = []
    for j, idx_ref in enumerate((idx0, idx1, idx2, idx3)):
        cps.append(pltpu.async_copy(seg_v.at[pl.ds(j * 128, 128)],
                                    shared.at[idx_ref], sem_a, add=True))
    for cp in cps:
        cp.wait()

    plsc.subcore_barrier()

    @pl.when(is_t0)
    def _():
        pltpu.sync_copy(shared, out_hbm)


def kernel(x, atomic_numbers, batch, shift):
    shift_p = jnp.zeros((128,), jnp.float32).at[:shift.shape[0]].set(shift[:, 0])
    x_p = jnp.zeros((_N_SEG,), jnp.float32).at[:x.shape[0]].set(x)
    out = _shift_kernel(x_p, atomic_numbers, batch, shift_p)
    return out[:x.shape[0]]
